# baseline (speedup 1.0000x reference)
import functools

import jax
import jax.numpy as jnp
from jax import lax
from jax.experimental import pallas as pl
from jax.experimental.pallas import tpu as pltpu
from jax.experimental.pallas import tpu_sc as plsc

N = 10000
E = 320000
D_IN = 128
D_HID = 16
D_OUT = 7
D_PAD = 16
D2 = 8

NC = 2
NS = 16
NW = NC * NS
CH = 128
NB = 4
SUPER = 4
GCH = SUPER * CH

N_PAD = 10240
RPT = N_PAD // NS

EPT = E // NW
NCHUNK = -(-EPT // CH)
NCHUNK = -(-NCHUNK // NB) * NB
EPT_P = NCHUNK * CH
GROUPS = NCHUNK // NB

_f32 = jnp.float32


def _sc_mesh():
    return plsc.VectorSubcoreMesh(
        core_axis_name="c", subcore_axis_name="s", num_cores=NC, num_subcores=NS
    )


def _sc_degree(dstg, zeros1d):
    @functools.partial(
        pl.kernel,
        out_type=jax.ShapeDtypeStruct((NC * N_PAD,), _f32),
        mesh=_sc_mesh(),
        scratch_types=[
            pltpu.VMEM((NCHUNK, CH), jnp.int32),
            pltpu.VMEM((CH,), _f32),
            pltpu.VMEM((RPT,), _f32),
            pltpu.VMEM_SHARED((N_PAD,), _f32),
            pltpu.SemaphoreType.DMA,
        ],
    )
    def k(dst_hbm, z_hbm, out_hbm, idx_d, ones_v, stage, acc, sem_s):
        cid = lax.axis_index("c")
        sid = lax.axis_index("s")
        wid = cid * NS + sid
        for j in range(CH // 16):
            ones_v[pl.ds(j * 16, 16)] = jnp.ones((16,), _f32)
        pltpu.sync_copy(dst_hbm.at[wid], idx_d)
        pltpu.sync_copy(z_hbm.at[pl.ds(0, RPT)], stage)
        pltpu.sync_copy(stage, acc.at[pl.ds(sid * RPT, RPT)])
        plsc.subcore_barrier()

        def body(g, carry):
            descs = []
            for b in range(NB):
                descs.append(pltpu.async_copy(
                    ones_v, acc.at[idx_d.at[g * NB + b]], sem_s, add=True))
            for d in descs:
                d.wait()
            return carry

        lax.fori_loop(0, GROUPS, body, 0)
        plsc.subcore_barrier()
        pltpu.sync_copy(acc.at[pl.ds(sid * RPT, RPT)], stage)
        pltpu.sync_copy(stage, out_hbm.at[pl.ds(cid * N_PAD + sid * RPT, RPT)])

    return k(dstg, zeros1d)


def _sc_scatter(hp, srcg, dstg, zeros2d, D):
    @functools.partial(
        pl.kernel,
        out_type=jax.ShapeDtypeStruct((NC, N_PAD, D), _f32),
        mesh=_sc_mesh(),
        compiler_params=pltpu.CompilerParams(use_tc_tiling_on_sc=False),
        scratch_types=[
            pltpu.VMEM((EPT_P,), jnp.int32),
            pltpu.VMEM((NCHUNK, CH), jnp.int32),
            [pltpu.VMEM((GCH, D), _f32)] * NB,
            pltpu.VMEM((RPT, D), _f32),
            pltpu.VMEM_SHARED((N_PAD, D), _f32),
            [pltpu.SemaphoreType.DMA] * NB,
            [pltpu.SemaphoreType.DMA] * NB,
        ],
    )
    def k(hp_hbm, src_hbm, dst_hbm, z_hbm, out_hbm, idx_s, idx_d, rows, stage,
          acc, sem_g, sem_s):
        cid = lax.axis_index("c")
        sid = lax.axis_index("s")
        wid = cid * NS + sid
        pltpu.sync_copy(src_hbm.at[wid], idx_s)
        pltpu.sync_copy(dst_hbm.at[wid], idx_d)
        pltpu.sync_copy(z_hbm.at[pl.ds(0, RPT)], stage)
        pltpu.sync_copy(stage, acc.at[pl.ds(sid * RPT, RPT)])
        plsc.subcore_barrier()

        def _drain_scatters(b):
            pltpu.make_async_copy(
                hp_hbm.at[pl.ds(0, GCH)], rows[b], sem_s[b]).wait()

        def body(g, carry):
            gd = []
            for b in range(NB):
                pl.when(g > 0)(lambda b=b: _drain_scatters(b))
                q = (g * NB + b) * GCH
                gd.append(pltpu.async_copy(
                    hp_hbm.at[idx_s.at[pl.ds(q, GCH)]], rows[b], sem_g[b]))
            for b in range(NB):
                gd[b].wait()
                for u in range(SUPER):
                    j = (g * NB + b) * SUPER + u
                    pltpu.async_copy(
                        rows[b].at[pl.ds(u * CH, CH)],
                        acc.at[idx_d.at[j]], sem_s[b], add=True)
            return carry

        lax.fori_loop(0, NCHUNK // (SUPER * NB), body, 0)
        for b in range(NB):
            _drain_scatters(b)
        plsc.subcore_barrier()
        pltpu.sync_copy(acc.at[pl.ds(sid * RPT, RPT)], stage)
        pltpu.sync_copy(stage, out_hbm.at[cid, pl.ds(sid * RPT, RPT)])

    return k(hp, srcg, dstg, zeros2d)


_BM = 2000


def _tc_lin1(x, W1, degT):
    def body(x_ref, w_ref, d_ref, o_ref):
        d = d_ref[...]
        dis = lax.rsqrt(1.0 + d[:, 0] + d[:, 1])[:, None]
        h = jnp.dot(x_ref[...], w_ref[...], preferred_element_type=_f32)
        o_ref[...] = h * dis

    return pl.pallas_call(
        body,
        grid=(N // _BM,),
        in_specs=[
            pl.BlockSpec((_BM, D_IN), lambda i: (i, 0)),
            pl.BlockSpec((D_IN, D_HID), lambda i: (0, 0)),
            pl.BlockSpec((_BM, NC), lambda i: (i, 0)),
        ],
        out_specs=pl.BlockSpec((_BM, D_HID), lambda i: (i, 0)),
        out_shape=jax.ShapeDtypeStruct((N, D_HID), _f32),
    )(x, W1, degT)


def _tc_mid(accp, h1p, degT, b1, W2p):
    def body(a_ref, h_ref, d_ref, b_ref, w_ref, o_ref):
        d = d_ref[...]
        dis = lax.rsqrt(1.0 + d[:, 0] + d[:, 1])[:, None]
        z = (a_ref[0] + a_ref[1] + h_ref[...]) * dis + b_ref[...]
        r = jnp.maximum(z, 0.0)
        o_ref[...] = jnp.dot(r, w_ref[...], preferred_element_type=_f32) * dis

    return pl.pallas_call(
        body,
        grid=(N // _BM,),
        in_specs=[
            pl.BlockSpec((NC, _BM, D_PAD), lambda i: (0, i, 0)),
            pl.BlockSpec((_BM, D_PAD), lambda i: (i, 0)),
            pl.BlockSpec((_BM, NC), lambda i: (i, 0)),
            pl.BlockSpec((1, D_HID), lambda i: (0, 0)),
            pl.BlockSpec((D_HID, D2), lambda i: (0, 0)),
        ],
        out_specs=pl.BlockSpec((_BM, D2), lambda i: (i, 0)),
        out_shape=jax.ShapeDtypeStruct((N, D2), _f32),
    )(accp, h1p, degT, b1, W2p)


def _tc_out(accp, h2p, degT, b2p):
    def body(a_ref, h_ref, d_ref, b_ref, o_ref):
        d = d_ref[...]
        dis = lax.rsqrt(1.0 + d[:, 0] + d[:, 1])[:, None]
        o = (a_ref[0] + a_ref[1] + h_ref[...]) * dis + b_ref[...]
        col = lax.broadcasted_iota(jnp.int32, o.shape, 1)
        o = jnp.where(col < D_OUT, o, -1e30)
        m = jnp.max(o, axis=1, keepdims=True)
        s = jnp.sum(jnp.exp(o - m), axis=1, keepdims=True)
        o_ref[...] = o - m - jnp.log(s)

    return pl.pallas_call(
        body,
        grid=(N // _BM,),
        in_specs=[
            pl.BlockSpec((NC, _BM, D2), lambda i: (0, i, 0)),
            pl.BlockSpec((_BM, D2), lambda i: (i, 0)),
            pl.BlockSpec((_BM, NC), lambda i: (i, 0)),
            pl.BlockSpec((1, D2), lambda i: (0, 0)),
        ],
        out_specs=pl.BlockSpec((_BM, D2), lambda i: (i, 0)),
        out_shape=jax.ShapeDtypeStruct((N, D2), _f32),
    )(accp, h2p, degT, b2p)


def kernel(x, edge_index, W1, b1, W2, b2):
    ei = edge_index.astype(jnp.int32)
    pad = EPT_P - EPT
    srcg = jnp.pad(ei[0].reshape(NW, EPT), ((0, 0), (0, pad)))
    dstg = jnp.pad(ei[1].reshape(NW, EPT), ((0, 0), (0, pad)),
                   constant_values=N).reshape(NW, NCHUNK, CH)

    zeros1d = jnp.zeros((N_PAD,), _f32)
    zeros2d = jnp.zeros((N_PAD, D_PAD), _f32)
    zeros2d8 = jnp.zeros((N_PAD, D2), _f32)
    W2p = jnp.zeros((D_HID, D2), _f32).at[:, :D_OUT].set(W2)
    b1r = b1.reshape(1, D_HID)
    b2p = jnp.zeros((1, D2), _f32).at[0, :D_OUT].set(b2)

    degp = _sc_degree(dstg, zeros1d).reshape(NC, N_PAD)
    degT = degp[:, :N].T

    h1p = _tc_lin1(x, W1, degT)
    acc1 = _sc_scatter(h1p, srcg, dstg, zeros2d, D_PAD)
    h2p = _tc_mid(acc1, h1p, degT, b1r, W2p)
    acc2 = _sc_scatter(h2p, srcg, dstg, zeros2d8, D2)
    out = _tc_out(acc2, h2p, degT, b2p)
    return out[:, :D_OUT]


# speedup vs baseline: 1.0066x; 1.0066x over previous
import functools

import jax
import jax.numpy as jnp
from jax import lax
from jax.experimental import pallas as pl
from jax.experimental.pallas import tpu as pltpu
from jax.experimental.pallas import tpu_sc as plsc

N = 10000
E = 320000
D_IN = 128
D_HID = 16
D_OUT = 7
D_PAD = 16
D2 = 8

NC = 2
NS = 16
NW = NC * NS
CH = 128
NB = 4
SUPER = 4
GCH = SUPER * CH

N_PAD = 10240
RPT = N_PAD // NS

EPT = E // NW
NCHUNK = -(-EPT // CH)
NCHUNK = -(-NCHUNK // NB) * NB
EPT_P = NCHUNK * CH
GROUPS = NCHUNK // NB

_f32 = jnp.float32


def _sc_mesh():
    return plsc.VectorSubcoreMesh(
        core_axis_name="c", subcore_axis_name="s", num_cores=NC, num_subcores=NS
    )


def _sc_degree(dstg, zeros1d):
    @functools.partial(
        pl.kernel,
        out_type=jax.ShapeDtypeStruct((NC * N_PAD,), _f32),
        mesh=_sc_mesh(),
        scratch_types=[
            pltpu.VMEM((NCHUNK, CH), jnp.int32),
            pltpu.VMEM((CH,), _f32),
            pltpu.VMEM((RPT,), _f32),
            pltpu.VMEM_SHARED((N_PAD,), _f32),
            pltpu.SemaphoreType.DMA,
        ],
    )
    def k(dst_hbm, z_hbm, out_hbm, idx_d, ones_v, stage, acc, sem_s):
        cid = lax.axis_index("c")
        sid = lax.axis_index("s")
        wid = cid * NS + sid
        for j in range(CH // 16):
            ones_v[pl.ds(j * 16, 16)] = jnp.ones((16,), _f32)
        pltpu.sync_copy(dst_hbm.at[wid], idx_d)
        pltpu.sync_copy(z_hbm.at[pl.ds(0, RPT)], stage)
        pltpu.sync_copy(stage, acc.at[pl.ds(sid * RPT, RPT)])
        plsc.subcore_barrier()

        def body(g, carry):
            for b in range(NB):
                pltpu.async_copy(
                    ones_v, acc.at[idx_d.at[g * NB + b]], sem_s, add=True)
            return carry

        lax.fori_loop(0, GROUPS, body, 0)

        def drain(g, carry):
            pltpu.make_async_copy(
                z_hbm.at[pl.ds(0, CH)], ones_v, sem_s).wait()
            return carry

        lax.fori_loop(0, NCHUNK, drain, 0)
        plsc.subcore_barrier()
        pltpu.sync_copy(acc.at[pl.ds(sid * RPT, RPT)], stage)
        pltpu.sync_copy(stage, out_hbm.at[pl.ds(cid * N_PAD + sid * RPT, RPT)])

    return k(dstg, zeros1d)


def _sc_scatter(hp, srcg, dstg, zeros2d, D):
    @functools.partial(
        pl.kernel,
        out_type=jax.ShapeDtypeStruct((NC, N_PAD, D), _f32),
        mesh=_sc_mesh(),
        compiler_params=pltpu.CompilerParams(use_tc_tiling_on_sc=False),
        scratch_types=[
            pltpu.VMEM((EPT_P,), jnp.int32),
            pltpu.VMEM((NCHUNK, CH), jnp.int32),
            [pltpu.VMEM((GCH, D), _f32)] * NB,
            pltpu.VMEM((RPT, D), _f32),
            pltpu.VMEM_SHARED((N_PAD, D), _f32),
            [pltpu.SemaphoreType.DMA] * NB,
            [pltpu.SemaphoreType.DMA] * NB,
        ],
    )
    def k(hp_hbm, src_hbm, dst_hbm, z_hbm, out_hbm, idx_s, idx_d, rows, stage,
          acc, sem_g, sem_s):
        cid = lax.axis_index("c")
        sid = lax.axis_index("s")
        wid = cid * NS + sid
        pltpu.sync_copy(src_hbm.at[wid], idx_s)
        pltpu.sync_copy(dst_hbm.at[wid], idx_d)
        pltpu.sync_copy(z_hbm.at[pl.ds(0, RPT)], stage)
        pltpu.sync_copy(stage, acc.at[pl.ds(sid * RPT, RPT)])
        plsc.subcore_barrier()

        def _drain_scatters(b):
            pltpu.make_async_copy(
                hp_hbm.at[pl.ds(0, GCH)], rows[b], sem_s[b]).wait()

        def body(g, carry):
            gd = []
            for b in range(NB):
                pl.when(g > 0)(lambda b=b: _drain_scatters(b))
                q = (g * NB + b) * GCH
                gd.append(pltpu.async_copy(
                    hp_hbm.at[idx_s.at[pl.ds(q, GCH)]], rows[b], sem_g[b]))
            for b in range(NB):
                gd[b].wait()
                for u in range(SUPER):
                    j = (g * NB + b) * SUPER + u
                    pltpu.async_copy(
                        rows[b].at[pl.ds(u * CH, CH)],
                        acc.at[idx_d.at[j]], sem_s[b], add=True)
            return carry

        lax.fori_loop(0, NCHUNK // (SUPER * NB), body, 0)
        for b in range(NB):
            _drain_scatters(b)
        plsc.subcore_barrier()
        pltpu.sync_copy(acc.at[pl.ds(sid * RPT, RPT)], stage)
        pltpu.sync_copy(stage, out_hbm.at[cid, pl.ds(sid * RPT, RPT)])

    return k(hp, srcg, dstg, zeros2d)


_BM = 2000


def _tc_lin1(x, W1, degT):
    def body(x_ref, w_ref, d_ref, o_ref):
        d = d_ref[...]
        dis = lax.rsqrt(1.0 + d[:, 0] + d[:, 1])[:, None]
        h = jnp.dot(x_ref[...], w_ref[...], preferred_element_type=_f32)
        o_ref[...] = h * dis

    return pl.pallas_call(
        body,
        grid=(N // _BM,),
        in_specs=[
            pl.BlockSpec((_BM, D_IN), lambda i: (i, 0)),
            pl.BlockSpec((D_IN, D_HID), lambda i: (0, 0)),
            pl.BlockSpec((_BM, NC), lambda i: (i, 0)),
        ],
        out_specs=pl.BlockSpec((_BM, D_HID), lambda i: (i, 0)),
        out_shape=jax.ShapeDtypeStruct((N, D_HID), _f32),
    )(x, W1, degT)


def _tc_mid(accp, h1p, degT, b1, W2p):
    def body(a_ref, h_ref, d_ref, b_ref, w_ref, o_ref):
        d = d_ref[...]
        dis = lax.rsqrt(1.0 + d[:, 0] + d[:, 1])[:, None]
        z = (a_ref[0] + a_ref[1] + h_ref[...]) * dis + b_ref[...]
        r = jnp.maximum(z, 0.0)
        o_ref[...] = jnp.dot(r, w_ref[...], preferred_element_type=_f32) * dis

    return pl.pallas_call(
        body,
        grid=(N // _BM,),
        in_specs=[
            pl.BlockSpec((NC, _BM, D_PAD), lambda i: (0, i, 0)),
            pl.BlockSpec((_BM, D_PAD), lambda i: (i, 0)),
            pl.BlockSpec((_BM, NC), lambda i: (i, 0)),
            pl.BlockSpec((1, D_HID), lambda i: (0, 0)),
            pl.BlockSpec((D_HID, D2), lambda i: (0, 0)),
        ],
        out_specs=pl.BlockSpec((_BM, D2), lambda i: (i, 0)),
        out_shape=jax.ShapeDtypeStruct((N, D2), _f32),
    )(accp, h1p, degT, b1, W2p)


def _tc_out(accp, h2p, degT, b2p):
    def body(a_ref, h_ref, d_ref, b_ref, o_ref):
        d = d_ref[...]
        dis = lax.rsqrt(1.0 + d[:, 0] + d[:, 1])[:, None]
        o = (a_ref[0] + a_ref[1] + h_ref[...]) * dis + b_ref[...]
        col = lax.broadcasted_iota(jnp.int32, o.shape, 1)
        o = jnp.where(col < D_OUT, o, -1e30)
        m = jnp.max(o, axis=1, keepdims=True)
        s = jnp.sum(jnp.exp(o - m), axis=1, keepdims=True)
        o_ref[...] = o - m - jnp.log(s)

    return pl.pallas_call(
        body,
        grid=(N // _BM,),
        in_specs=[
            pl.BlockSpec((NC, _BM, D2), lambda i: (0, i, 0)),
            pl.BlockSpec((_BM, D2), lambda i: (i, 0)),
            pl.BlockSpec((_BM, NC), lambda i: (i, 0)),
            pl.BlockSpec((1, D2), lambda i: (0, 0)),
        ],
        out_specs=pl.BlockSpec((_BM, D2), lambda i: (i, 0)),
        out_shape=jax.ShapeDtypeStruct((N, D2), _f32),
    )(accp, h2p, degT, b2p)


def kernel(x, edge_index, W1, b1, W2, b2):
    ei = edge_index.astype(jnp.int32)
    pad = EPT_P - EPT
    srcg = jnp.pad(ei[0].reshape(NW, EPT), ((0, 0), (0, pad)))
    dstg = jnp.pad(ei[1].reshape(NW, EPT), ((0, 0), (0, pad)),
                   constant_values=N).reshape(NW, NCHUNK, CH)

    zeros1d = jnp.zeros((N_PAD,), _f32)
    zeros2d = jnp.zeros((N_PAD, D_PAD), _f32)
    zeros2d8 = jnp.zeros((N_PAD, D2), _f32)
    W2p = jnp.zeros((D_HID, D2), _f32).at[:, :D_OUT].set(W2)
    b1r = b1.reshape(1, D_HID)
    b2p = jnp.zeros((1, D2), _f32).at[0, :D_OUT].set(b2)

    degp = _sc_degree(dstg, zeros1d).reshape(NC, N_PAD)
    degT = degp[:, :N].T

    h1p = _tc_lin1(x, W1, degT)
    acc1 = _sc_scatter(h1p, srcg, dstg, zeros2d, D_PAD)
    h2p = _tc_mid(acc1, h1p, degT, b1r, W2p)
    acc2 = _sc_scatter(h2p, srcg, dstg, zeros2d8, D2)
    out = _tc_out(acc2, h2p, degT, b2p)
    return out[:, :D_OUT]


# speedup vs baseline: 1.0156x; 1.0089x over previous
import functools

import jax
import jax.numpy as jnp
from jax import lax
from jax.experimental import pallas as pl
from jax.experimental.pallas import tpu as pltpu
from jax.experimental.pallas import tpu_sc as plsc

N = 10000
E = 320000
D_IN = 128
D_HID = 16
D_OUT = 7
D_PAD = 16
D2 = 8

NC = 2
NS = 16
NW = NC * NS
CH = 128
NB = 4
SUPER = 4
GCH = SUPER * CH

N_PAD = 10240
RPT = N_PAD // NS

EPT = E // NW
NCHUNK = -(-EPT // CH)
NCHUNK = -(-NCHUNK // NB) * NB
EPT_P = NCHUNK * CH
GROUPS = NCHUNK // NB

_f32 = jnp.float32


def _sc_mesh():
    return plsc.VectorSubcoreMesh(
        core_axis_name="c", subcore_axis_name="s", num_cores=NC, num_subcores=NS
    )


def _sc_degree(dstg, zeros1d):
    @functools.partial(
        pl.kernel,
        out_type=jax.ShapeDtypeStruct((NC * N_PAD,), _f32),
        mesh=_sc_mesh(),
        scratch_types=[
            pltpu.VMEM((NCHUNK, CH), jnp.int32),
            pltpu.VMEM((CH,), _f32),
            pltpu.VMEM((RPT,), _f32),
            pltpu.VMEM_SHARED((N_PAD,), _f32),
            pltpu.SemaphoreType.DMA,
        ],
    )
    def k(dst_hbm, z_hbm, out_hbm, idx_d, ones_v, stage, acc, sem_s):
        cid = lax.axis_index("c")
        sid = lax.axis_index("s")
        wid = cid * NS + sid
        for j in range(CH // 16):
            ones_v[pl.ds(j * 16, 16)] = jnp.ones((16,), _f32)
        pltpu.sync_copy(dst_hbm.at[wid], idx_d)
        pltpu.sync_copy(z_hbm.at[pl.ds(0, RPT)], stage)
        pltpu.sync_copy(stage, acc.at[pl.ds(sid * RPT, RPT)])
        plsc.subcore_barrier()

        def body(g, carry):
            for b in range(NB):
                pltpu.async_copy(
                    ones_v, acc.at[idx_d.at[g * NB + b]], sem_s, add=True)
            return carry

        lax.fori_loop(0, GROUPS, body, 0)

        def drain(g, carry):
            pltpu.make_async_copy(
                z_hbm.at[pl.ds(0, CH)], ones_v, sem_s).wait()
            return carry

        lax.fori_loop(0, NCHUNK, drain, 0)
        plsc.subcore_barrier()
        pltpu.sync_copy(acc.at[pl.ds(sid * RPT, RPT)], stage)
        pltpu.sync_copy(stage, out_hbm.at[pl.ds(cid * N_PAD + sid * RPT, RPT)])

    return k(dstg, zeros1d)


def _sc_scatter(hp, srcg, dstg, zeros2d, D):
    @functools.partial(
        pl.kernel,
        out_type=jax.ShapeDtypeStruct((NC, N_PAD, D), _f32),
        mesh=_sc_mesh(),
        compiler_params=pltpu.CompilerParams(use_tc_tiling_on_sc=False),
        scratch_types=[
            pltpu.VMEM((EPT_P,), jnp.int32),
            pltpu.VMEM((NCHUNK, CH), jnp.int32),
            [pltpu.VMEM((GCH, D), _f32)] * NB,
            pltpu.VMEM((RPT, D), _f32),
            pltpu.VMEM_SHARED((N_PAD, D), _f32),
            [pltpu.SemaphoreType.DMA] * NB,
            [pltpu.SemaphoreType.DMA] * NB,
        ],
    )
    def k(hp_hbm, src_hbm, dst_hbm, z_hbm, out_hbm, idx_s, idx_d, rows, stage,
          acc, sem_g, sem_s):
        cid = lax.axis_index("c")
        sid = lax.axis_index("s")
        wid = cid * NS + sid
        d1 = pltpu.async_copy(src_hbm.at[wid], idx_s, sem_g[0])
        d2 = pltpu.async_copy(dst_hbm.at[wid], idx_d, sem_g[1])
        d3 = pltpu.async_copy(z_hbm.at[pl.ds(0, RPT)], stage, sem_g[2])
        d3.wait()
        d4 = pltpu.async_copy(stage, acc.at[pl.ds(sid * RPT, RPT)], sem_g[3])
        d1.wait()
        d2.wait()
        d4.wait()
        plsc.subcore_barrier()

        def _drain_scatters(b):
            pltpu.make_async_copy(
                hp_hbm.at[pl.ds(0, GCH)], rows[b], sem_s[b]).wait()

        def body(g, carry):
            gd = []
            for b in range(NB):
                pl.when(g > 0)(lambda b=b: _drain_scatters(b))
                q = (g * NB + b) * GCH
                gd.append(pltpu.async_copy(
                    hp_hbm.at[idx_s.at[pl.ds(q, GCH)]], rows[b], sem_g[b]))
            for b in range(NB):
                gd[b].wait()
                for u in range(SUPER):
                    j = (g * NB + b) * SUPER + u
                    pltpu.async_copy(
                        rows[b].at[pl.ds(u * CH, CH)],
                        acc.at[idx_d.at[j]], sem_s[b], add=True)
            return carry

        lax.fori_loop(0, NCHUNK // (SUPER * NB), body, 0)
        for b in range(NB):
            _drain_scatters(b)
        plsc.subcore_barrier()
        pltpu.sync_copy(acc.at[pl.ds(sid * RPT, RPT)], stage)
        pltpu.sync_copy(stage, out_hbm.at[cid, pl.ds(sid * RPT, RPT)])

    return k(hp, srcg, dstg, zeros2d)


_BM = 2000


def _tc_lin1(x, W1, degT):
    def body(x_ref, w_ref, d_ref, o_ref):
        d = d_ref[...]
        dis = lax.rsqrt(1.0 + d[:, 0] + d[:, 1])[:, None]
        h = jnp.dot(x_ref[...], w_ref[...], preferred_element_type=_f32)
        o_ref[...] = h * dis

    return pl.pallas_call(
        body,
        grid=(N // _BM,),
        in_specs=[
            pl.BlockSpec((_BM, D_IN), lambda i: (i, 0)),
            pl.BlockSpec((D_IN, D_HID), lambda i: (0, 0)),
            pl.BlockSpec((_BM, NC), lambda i: (i, 0)),
        ],
        out_specs=pl.BlockSpec((_BM, D_HID), lambda i: (i, 0)),
        out_shape=jax.ShapeDtypeStruct((N, D_HID), _f32),
    )(x, W1, degT)


def _tc_mid(accp, h1p, degT, b1, W2p):
    def body(a_ref, h_ref, d_ref, b_ref, w_ref, o_ref):
        d = d_ref[...]
        dis = lax.rsqrt(1.0 + d[:, 0] + d[:, 1])[:, None]
        z = (a_ref[0] + a_ref[1] + h_ref[...]) * dis + b_ref[...]
        r = jnp.maximum(z, 0.0)
        o_ref[...] = jnp.dot(r, w_ref[...], preferred_element_type=_f32) * dis

    return pl.pallas_call(
        body,
        grid=(N // _BM,),
        in_specs=[
            pl.BlockSpec((NC, _BM, D_PAD), lambda i: (0, i, 0)),
            pl.BlockSpec((_BM, D_PAD), lambda i: (i, 0)),
            pl.BlockSpec((_BM, NC), lambda i: (i, 0)),
            pl.BlockSpec((1, D_HID), lambda i: (0, 0)),
            pl.BlockSpec((D_HID, D2), lambda i: (0, 0)),
        ],
        out_specs=pl.BlockSpec((_BM, D2), lambda i: (i, 0)),
        out_shape=jax.ShapeDtypeStruct((N, D2), _f32),
    )(accp, h1p, degT, b1, W2p)


def _tc_out(accp, h2p, degT, b2p):
    def body(a_ref, h_ref, d_ref, b_ref, o_ref):
        d = d_ref[...]
        dis = lax.rsqrt(1.0 + d[:, 0] + d[:, 1])[:, None]
        o = (a_ref[0] + a_ref[1] + h_ref[...]) * dis + b_ref[...]
        col = lax.broadcasted_iota(jnp.int32, o.shape, 1)
        o = jnp.where(col < D_OUT, o, -1e30)
        m = jnp.max(o, axis=1, keepdims=True)
        s = jnp.sum(jnp.exp(o - m), axis=1, keepdims=True)
        o_ref[...] = o - m - jnp.log(s)

    return pl.pallas_call(
        body,
        grid=(N // _BM,),
        in_specs=[
            pl.BlockSpec((NC, _BM, D2), lambda i: (0, i, 0)),
            pl.BlockSpec((_BM, D2), lambda i: (i, 0)),
            pl.BlockSpec((_BM, NC), lambda i: (i, 0)),
            pl.BlockSpec((1, D2), lambda i: (0, 0)),
        ],
        out_specs=pl.BlockSpec((_BM, D2), lambda i: (i, 0)),
        out_shape=jax.ShapeDtypeStruct((N, D2), _f32),
    )(accp, h2p, degT, b2p)


def kernel(x, edge_index, W1, b1, W2, b2):
    ei = edge_index.astype(jnp.int32)
    pad = EPT_P - EPT
    srcg = jnp.pad(ei[0].reshape(NW, EPT), ((0, 0), (0, pad)))
    dstg = jnp.pad(ei[1].reshape(NW, EPT), ((0, 0), (0, pad)),
                   constant_values=N).reshape(NW, NCHUNK, CH)

    zeros1d = jnp.zeros((N_PAD,), _f32)
    zeros2d = jnp.zeros((N_PAD, D_PAD), _f32)
    zeros2d8 = jnp.zeros((N_PAD, D2), _f32)
    W2p = jnp.zeros((D_HID, D2), _f32).at[:, :D_OUT].set(W2)
    b1r = b1.reshape(1, D_HID)
    b2p = jnp.zeros((1, D2), _f32).at[0, :D_OUT].set(b2)

    degp = _sc_degree(dstg, zeros1d).reshape(NC, N_PAD)
    degT = degp[:, :N].T

    h1p = _tc_lin1(x, W1, degT)
    acc1 = _sc_scatter(h1p, srcg, dstg, zeros2d, D_PAD)
    h2p = _tc_mid(acc1, h1p, degT, b1r, W2p)
    acc2 = _sc_scatter(h2p, srcg, dstg, zeros2d8, D2)
    out = _tc_out(acc2, h2p, degT, b2p)
    return out[:, :D_OUT]


# speedup vs baseline: 1.0161x; 1.0006x over previous
import functools

import jax
import jax.numpy as jnp
from jax import lax
from jax.experimental import pallas as pl
from jax.experimental.pallas import tpu as pltpu
from jax.experimental.pallas import tpu_sc as plsc

N = 10000
E = 320000
D_IN = 128
D_HID = 16
D_OUT = 7
D_PAD = 16
D2 = 8

NC = 2
NS = 16
NW = NC * NS
CH = 128
NB = 4
SUPER = 4
GCH = SUPER * CH

N_PAD = 10240
RPT = N_PAD // NS

EPT = E // NW
NCHUNK = -(-EPT // CH)
NCHUNK = -(-NCHUNK // NB) * NB
EPT_P = NCHUNK * CH
GROUPS = NCHUNK // NB

_f32 = jnp.float32


def _sc_mesh():
    return plsc.VectorSubcoreMesh(
        core_axis_name="c", subcore_axis_name="s", num_cores=NC, num_subcores=NS
    )


def _sc_degree(dstg, zeros1d):
    @functools.partial(
        pl.kernel,
        out_type=jax.ShapeDtypeStruct((NC * N_PAD,), _f32),
        mesh=_sc_mesh(),
        scratch_types=[
            pltpu.VMEM((NCHUNK, CH), jnp.int32),
            pltpu.VMEM((CH,), _f32),
            pltpu.VMEM((RPT,), _f32),
            pltpu.VMEM_SHARED((N_PAD,), _f32),
            pltpu.SemaphoreType.DMA,
            pltpu.SemaphoreType.DMA,
        ],
    )
    def k(dst_hbm, z_hbm, out_hbm, idx_d, ones_v, stage, acc, sem_s, sem_p):
        cid = lax.axis_index("c")
        sid = lax.axis_index("s")
        wid = cid * NS + sid
        for j in range(CH // 16):
            ones_v[pl.ds(j * 16, 16)] = jnp.ones((16,), _f32)
        d1 = pltpu.async_copy(dst_hbm.at[wid], idx_d, sem_p)
        d2 = pltpu.async_copy(z_hbm.at[pl.ds(0, RPT)], stage, sem_s)
        d2.wait()
        d3 = pltpu.async_copy(stage, acc.at[pl.ds(sid * RPT, RPT)], sem_s)
        d1.wait()
        d3.wait()
        plsc.subcore_barrier()

        def body(g, carry):
            for b in range(NB):
                pltpu.async_copy(
                    ones_v, acc.at[idx_d.at[g * NB + b]], sem_s, add=True)
            return carry

        lax.fori_loop(0, GROUPS, body, 0)

        def drain(g, carry):
            pltpu.make_async_copy(
                z_hbm.at[pl.ds(0, CH)], ones_v, sem_s).wait()
            return carry

        lax.fori_loop(0, NCHUNK, drain, 0)
        plsc.subcore_barrier()
        pltpu.sync_copy(acc.at[pl.ds(sid * RPT, RPT)], stage)
        pltpu.sync_copy(stage, out_hbm.at[pl.ds(cid * N_PAD + sid * RPT, RPT)])

    return k(dstg, zeros1d)


def _sc_scatter(hp, srcg, dstg, zeros2d, D):
    @functools.partial(
        pl.kernel,
        out_type=jax.ShapeDtypeStruct((NC, N_PAD, D), _f32),
        mesh=_sc_mesh(),
        compiler_params=pltpu.CompilerParams(use_tc_tiling_on_sc=False),
        scratch_types=[
            pltpu.VMEM((EPT_P,), jnp.int32),
            pltpu.VMEM((NCHUNK, CH), jnp.int32),
            [pltpu.VMEM((GCH, D), _f32)] * NB,
            pltpu.VMEM((RPT, D), _f32),
            pltpu.VMEM_SHARED((N_PAD, D), _f32),
            [pltpu.SemaphoreType.DMA] * NB,
            [pltpu.SemaphoreType.DMA] * NB,
        ],
    )
    def k(hp_hbm, src_hbm, dst_hbm, z_hbm, out_hbm, idx_s, idx_d, rows, stage,
          acc, sem_g, sem_s):
        cid = lax.axis_index("c")
        sid = lax.axis_index("s")
        wid = cid * NS + sid
        d1 = pltpu.async_copy(src_hbm.at[wid], idx_s, sem_g[0])
        d2 = pltpu.async_copy(dst_hbm.at[wid], idx_d, sem_g[1])
        d3 = pltpu.async_copy(z_hbm.at[pl.ds(0, RPT)], stage, sem_g[2])
        d3.wait()
        d4 = pltpu.async_copy(stage, acc.at[pl.ds(sid * RPT, RPT)], sem_g[3])
        d1.wait()
        d2.wait()
        d4.wait()
        plsc.subcore_barrier()

        def _drain_scatters(b):
            pltpu.make_async_copy(
                hp_hbm.at[pl.ds(0, GCH)], rows[b], sem_s[b]).wait()

        def body(g, carry):
            gd = []
            for b in range(NB):
                pl.when(g > 0)(lambda b=b: _drain_scatters(b))
                q = (g * NB + b) * GCH
                gd.append(pltpu.async_copy(
                    hp_hbm.at[idx_s.at[pl.ds(q, GCH)]], rows[b], sem_g[b]))
            for b in range(NB):
                gd[b].wait()
                for u in range(SUPER):
                    j = (g * NB + b) * SUPER + u
                    pltpu.async_copy(
                        rows[b].at[pl.ds(u * CH, CH)],
                        acc.at[idx_d.at[j]], sem_s[b], add=True)
            return carry

        lax.fori_loop(0, NCHUNK // (SUPER * NB), body, 0)
        for b in range(NB):
            _drain_scatters(b)
        plsc.subcore_barrier()
        pltpu.sync_copy(acc.at[pl.ds(sid * RPT, RPT)], stage)
        pltpu.sync_copy(stage, out_hbm.at[cid, pl.ds(sid * RPT, RPT)])

    return k(hp, srcg, dstg, zeros2d)


_BM = 2000


def _tc_lin1(x, W1, degT):
    def body(x_ref, w_ref, d_ref, o_ref):
        d = d_ref[...]
        dis = lax.rsqrt(1.0 + d[:, 0] + d[:, 1])[:, None]
        h = jnp.dot(x_ref[...], w_ref[...], preferred_element_type=_f32)
        o_ref[...] = h * dis

    return pl.pallas_call(
        body,
        grid=(N // _BM,),
        in_specs=[
            pl.BlockSpec((_BM, D_IN), lambda i: (i, 0)),
            pl.BlockSpec((D_IN, D_HID), lambda i: (0, 0)),
            pl.BlockSpec((_BM, NC), lambda i: (i, 0)),
        ],
        out_specs=pl.BlockSpec((_BM, D_HID), lambda i: (i, 0)),
        out_shape=jax.ShapeDtypeStruct((N, D_HID), _f32),
    )(x, W1, degT)


def _tc_mid(accp, h1p, degT, b1, W2p):
    def body(a_ref, h_ref, d_ref, b_ref, w_ref, o_ref):
        d = d_ref[...]
        dis = lax.rsqrt(1.0 + d[:, 0] + d[:, 1])[:, None]
        z = (a_ref[0] + a_ref[1] + h_ref[...]) * dis + b_ref[...]
        r = jnp.maximum(z, 0.0)
        o_ref[...] = jnp.dot(r, w_ref[...], preferred_element_type=_f32) * dis

    return pl.pallas_call(
        body,
        grid=(N // _BM,),
        in_specs=[
            pl.BlockSpec((NC, _BM, D_PAD), lambda i: (0, i, 0)),
            pl.BlockSpec((_BM, D_PAD), lambda i: (i, 0)),
            pl.BlockSpec((_BM, NC), lambda i: (i, 0)),
            pl.BlockSpec((1, D_HID), lambda i: (0, 0)),
            pl.BlockSpec((D_HID, D2), lambda i: (0, 0)),
        ],
        out_specs=pl.BlockSpec((_BM, D2), lambda i: (i, 0)),
        out_shape=jax.ShapeDtypeStruct((N, D2), _f32),
    )(accp, h1p, degT, b1, W2p)


def _tc_out(accp, h2p, degT, b2p):
    def body(a_ref, h_ref, d_ref, b_ref, o_ref):
        d = d_ref[...]
        dis = lax.rsqrt(1.0 + d[:, 0] + d[:, 1])[:, None]
        o = (a_ref[0] + a_ref[1] + h_ref[...]) * dis + b_ref[...]
        col = lax.broadcasted_iota(jnp.int32, o.shape, 1)
        o = jnp.where(col < D_OUT, o, -1e30)
        m = jnp.max(o, axis=1, keepdims=True)
        s = jnp.sum(jnp.exp(o - m), axis=1, keepdims=True)
        o_ref[...] = o - m - jnp.log(s)

    return pl.pallas_call(
        body,
        grid=(N // _BM,),
        in_specs=[
            pl.BlockSpec((NC, _BM, D2), lambda i: (0, i, 0)),
            pl.BlockSpec((_BM, D2), lambda i: (i, 0)),
            pl.BlockSpec((_BM, NC), lambda i: (i, 0)),
            pl.BlockSpec((1, D2), lambda i: (0, 0)),
        ],
        out_specs=pl.BlockSpec((_BM, D2), lambda i: (i, 0)),
        out_shape=jax.ShapeDtypeStruct((N, D2), _f32),
    )(accp, h2p, degT, b2p)


def kernel(x, edge_index, W1, b1, W2, b2):
    ei = edge_index.astype(jnp.int32)
    pad = EPT_P - EPT
    srcg = jnp.pad(ei[0].reshape(NW, EPT), ((0, 0), (0, pad)))
    dstg = jnp.pad(ei[1].reshape(NW, EPT), ((0, 0), (0, pad)),
                   constant_values=N).reshape(NW, NCHUNK, CH)

    zeros1d = jnp.zeros((N_PAD,), _f32)
    zeros2d = jnp.zeros((N_PAD, D_PAD), _f32)
    zeros2d8 = jnp.zeros((N_PAD, D2), _f32)
    W2p = jnp.zeros((D_HID, D2), _f32).at[:, :D_OUT].set(W2)
    b1r = b1.reshape(1, D_HID)
    b2p = jnp.zeros((1, D2), _f32).at[0, :D_OUT].set(b2)

    degp = _sc_degree(dstg, zeros1d).reshape(NC, N_PAD)
    degT = degp[:, :N].T

    h1p = _tc_lin1(x, W1, degT)
    acc1 = _sc_scatter(h1p, srcg, dstg, zeros2d, D_PAD)
    h2p = _tc_mid(acc1, h1p, degT, b1r, W2p)
    acc2 = _sc_scatter(h2p, srcg, dstg, zeros2d8, D2)
    out = _tc_out(acc2, h2p, degT, b2p)
    return out[:, :D_OUT]
